# Initial kernel scaffold; baseline (speedup 1.0000x reference)
import jax
import jax.numpy as jnp
from jax.experimental import pallas as pl
from jax.experimental.pallas import tpu as pltpu

GAMMA = 0.1
BPARAM = 10.0
TAO = 1.0
BETA = 1.0
LAMBDA_ = 0.1
EPS = 1e-6
C = 16
P = 512
D = 64
BATCH = 1024


def _assign_tc_kernel(labels_ref, feat_ref, protos_in, counts_in,
                      protos_out, counts_out):
    protos_out[...] = protos_in[...]
    counts_out[...] = counts_in[...]
    iota = jax.lax.broadcasted_iota(jnp.int32, (P, 1), 0)

    def body(i, carry):
        lab = labels_ref[i]
        frow = feat_ref[pl.ds(i, 1), :]
        cp = protos_out[lab]
        diff = frow - cp + EPS
        sq = jnp.sum(diff * diff, axis=1, keepdims=True)
        sq = jnp.maximum(sq, 1e-12)
        minval = jnp.min(sq)
        idx = jnp.min(jnp.where(sq == minval, iota, P))
        onehot = iota == idx
        cnt_col = counts_out[lab]
        cval = jnp.sum(jnp.where(onehot, cnt_col, 0.0))
        psel = jnp.sum(jnp.where(onehot, cp, 0.0), axis=0, keepdims=True)
        newp = (psel * cval + frow) / (cval + 1.0)
        protos_out[lab] = jnp.where(onehot, newp, cp)
        counts_out[lab] = cnt_col + jnp.where(onehot, 1.0, 0.0)
        return carry

    jax.lax.fori_loop(0, BATCH, body, 0)


def _loss_tc_kernel(labels_ref, feat_ref, protos_ref,
                    out_ref, one_acc, num_acc, pw_acc):
    c = pl.program_id(0)

    @pl.when(c == 0)
    def _init():
        one_acc[...] = jnp.zeros_like(one_acc)
        num_acc[...] = jnp.zeros_like(num_acc)
        pw_acc[...] = jnp.zeros_like(pw_acc)

    pb = protos_ref[0]
    feats = feat_ref[...]
    dn = (((1,), (1,)), ((), ()))
    xy = jax.lax.dot_general(feats, pb, dn,
                             preferred_element_type=jnp.float32)
    ones_row = jnp.ones((1, D), jnp.float32)
    ynrow = jax.lax.dot_general(ones_row, pb * pb, dn,
                                preferred_element_type=jnp.float32)
    ysrow = jax.lax.dot_general(ones_row, pb, dn,
                                preferred_element_type=jnp.float32)
    xn = jnp.sum(feats * feats, axis=1, keepdims=True)
    xs = jnp.sum(feats, axis=1, keepdims=True)
    sq = xn + ynrow - 2.0 * xy + 2.0 * EPS * (xs - ysrow) + D * EPS * EPS
    sq = jnp.maximum(sq, 1e-12)
    expterm = jnp.exp(-GAMMA * sq)
    pc = jnp.sum(expterm, axis=1, keepdims=True)
    lab = labels_ref[...]
    mask = lab == c
    one_acc[...] += pc
    num_acc[...] += jnp.where(mask, pc, 0.0)
    dmin = jnp.sqrt(jnp.min(sq, axis=1, keepdims=True))
    sign = jnp.where(mask, 1.0, -1.0)
    z = BPARAM - (TAO - dmin) * sign
    soft = jnp.log(1.0 + jnp.exp(BETA * jnp.minimum(z, 10.0))) / BETA
    pw_acc[...] += jnp.where(z > 10.0, z, soft)

    @pl.when(c == C - 1)
    def _fin():
        one = one_acc[...]
        num = num_acc[...]
        safe = jnp.where(one > 0.0, one, 1.0)
        prob = jnp.where(one > 0.0, 1e-6 + num / safe, 1e-6 + one)
        dce = jnp.sum(-jnp.log(prob))
        pw = jnp.sum(pw_acc[...])
        out_ref[0, 0] = dce + LAMBDA_ * pw


def _assign(features, labels, prototypes, counts3, interpret=False):
    return pl.pallas_call(
        _assign_tc_kernel,
        out_shape=[
            jax.ShapeDtypeStruct((C, P, D), jnp.float32),
            jax.ShapeDtypeStruct((C, P, 1), jnp.float32),
        ],
        in_specs=[
            pl.BlockSpec(memory_space=pltpu.SMEM),
            pl.BlockSpec(memory_space=pltpu.ANY),
            pl.BlockSpec(memory_space=pltpu.ANY),
            pl.BlockSpec(memory_space=pltpu.ANY),
        ],
        out_specs=[
            pl.BlockSpec(memory_space=pltpu.ANY),
            pl.BlockSpec(memory_space=pltpu.ANY),
        ],
        interpret=interpret,
    )(labels, features, prototypes, counts3)


def _loss(labels2d, features, protos, interpret=False):
    return pl.pallas_call(
        _loss_tc_kernel,
        grid=(C,),
        out_shape=jax.ShapeDtypeStruct((1, 1), jnp.float32),
        in_specs=[
            pl.BlockSpec((BATCH, 1), lambda c: (0, 0)),
            pl.BlockSpec((BATCH, D), lambda c: (0, 0)),
            pl.BlockSpec((1, P, D), lambda c: (c, 0, 0)),
        ],
        out_specs=pl.BlockSpec((1, 1), lambda c: (0, 0)),
        scratch_shapes=[
            pltpu.VMEM((BATCH, 1), jnp.float32),
            pltpu.VMEM((BATCH, 1), jnp.float32),
            pltpu.VMEM((BATCH, 1), jnp.float32),
        ],
        interpret=interpret,
    )(labels2d, features, protos)


def kernel(features, labels, prototypes, counts):
    labels = labels.astype(jnp.int32)
    protos_up, _ = _assign(features, labels, prototypes, counts[..., None])
    out = _loss(labels[:, None], features, protos_up)
    return out[0, 0]


# speedup vs baseline: 23.3164x; 23.3164x over previous
import jax
import jax.numpy as jnp
from jax.experimental import pallas as pl
from jax.experimental.pallas import tpu as pltpu

GAMMA = 0.1
BPARAM = 10.0
TAO = 1.0
BETA = 1.0
LAMBDA_ = 0.1
EPS = 1e-6
C = 16
P = 512
D = 64
BATCH = 1024


def _assign_tc_kernel(labels_ref, feat_ref, protos_in, counts_in,
                      protos_out, counts_out):
    protos_out[...] = protos_in[...]
    counts_out[...] = counts_in[...]
    iota = jax.lax.broadcasted_iota(jnp.int32, (P, 1), 0)

    def body(i, carry):
        lab = labels_ref[i]
        frow = feat_ref[pl.ds(i, 1), :]
        cp = protos_out[lab]
        diff = frow - cp + EPS
        sq = jnp.sum(diff * diff, axis=1, keepdims=True)
        sq = jnp.maximum(sq, 1e-12)
        minval = jnp.min(sq)
        idx = jnp.min(jnp.where(sq == minval, iota, P))
        onehot = iota == idx
        cnt_col = counts_out[lab]
        cval = jnp.sum(jnp.where(onehot, cnt_col, 0.0))
        psel = jnp.sum(jnp.where(onehot, cp, 0.0), axis=0, keepdims=True)
        newp = (psel * cval + frow) / (cval + 1.0)
        protos_out[lab] = jnp.where(onehot, newp, cp)
        counts_out[lab] = cnt_col + jnp.where(onehot, 1.0, 0.0)
        return carry

    jax.lax.fori_loop(0, BATCH, body, 0)


def _loss_tc_kernel(labels_ref, feat_ref, protos_ref,
                    out_ref, one_acc, num_acc, pw_acc):
    c = pl.program_id(0)

    @pl.when(c == 0)
    def _init():
        one_acc[...] = jnp.zeros_like(one_acc)
        num_acc[...] = jnp.zeros_like(num_acc)
        pw_acc[...] = jnp.zeros_like(pw_acc)

    pb = protos_ref[0]
    feats = feat_ref[...]
    dn = (((1,), (1,)), ((), ()))
    xy = jax.lax.dot_general(feats, pb, dn,
                             preferred_element_type=jnp.float32)
    ones_row = jnp.ones((1, D), jnp.float32)
    ynrow = jax.lax.dot_general(ones_row, pb * pb, dn,
                                preferred_element_type=jnp.float32)
    ysrow = jax.lax.dot_general(ones_row, pb, dn,
                                preferred_element_type=jnp.float32)
    xn = jnp.sum(feats * feats, axis=1, keepdims=True)
    xs = jnp.sum(feats, axis=1, keepdims=True)
    sq = xn + ynrow - 2.0 * xy + 2.0 * EPS * (xs - ysrow) + D * EPS * EPS
    sq = jnp.maximum(sq, 1e-12)
    expterm = jnp.exp(-GAMMA * sq)
    pc = jnp.sum(expterm, axis=1, keepdims=True)
    lab = labels_ref[...]
    mask = lab == c
    one_acc[...] += pc
    num_acc[...] += jnp.where(mask, pc, 0.0)
    dmin = jnp.sqrt(jnp.min(sq, axis=1, keepdims=True))
    sign = jnp.where(mask, 1.0, -1.0)
    z = BPARAM - (TAO - dmin) * sign
    soft = jnp.log(1.0 + jnp.exp(BETA * jnp.minimum(z, 10.0))) / BETA
    pw_acc[...] += jnp.where(z > 10.0, z, soft)

    @pl.when(c == C - 1)
    def _fin():
        one = one_acc[...]
        num = num_acc[...]
        safe = jnp.where(one > 0.0, one, 1.0)
        prob = jnp.where(one > 0.0, 1e-6 + num / safe, 1e-6 + one)
        dce = jnp.sum(-jnp.log(prob))
        pw = jnp.sum(pw_acc[...])
        out_ref[...] = jnp.reshape(dce + LAMBDA_ * pw, (1, 1))


def _assign(features, labels, prototypes, counts3, interpret=False):
    return pl.pallas_call(
        _assign_tc_kernel,
        out_shape=[
            jax.ShapeDtypeStruct((C, P, D), jnp.float32),
            jax.ShapeDtypeStruct((C, P, 1), jnp.float32),
        ],
        in_specs=[
            pl.BlockSpec(memory_space=pltpu.SMEM),
            pl.BlockSpec(memory_space=pltpu.VMEM),
            pl.BlockSpec(memory_space=pltpu.VMEM),
            pl.BlockSpec(memory_space=pltpu.VMEM),
        ],
        out_specs=[
            pl.BlockSpec(memory_space=pltpu.VMEM),
            pl.BlockSpec(memory_space=pltpu.VMEM),
        ],
        interpret=interpret,
    )(labels, features, prototypes, counts3)


def _loss(labels2d, features, protos, interpret=False):
    return pl.pallas_call(
        _loss_tc_kernel,
        grid=(C,),
        out_shape=jax.ShapeDtypeStruct((1, 1), jnp.float32),
        in_specs=[
            pl.BlockSpec((BATCH, 1), lambda c: (0, 0)),
            pl.BlockSpec((BATCH, D), lambda c: (0, 0)),
            pl.BlockSpec((1, P, D), lambda c: (c, 0, 0)),
        ],
        out_specs=pl.BlockSpec((1, 1), lambda c: (0, 0)),
        scratch_shapes=[
            pltpu.VMEM((BATCH, 1), jnp.float32),
            pltpu.VMEM((BATCH, 1), jnp.float32),
            pltpu.VMEM((BATCH, 1), jnp.float32),
        ],
        interpret=interpret,
    )(labels2d, features, protos)


def kernel(features, labels, prototypes, counts):
    labels = labels.astype(jnp.int32)
    protos_up, _ = _assign(features, labels, prototypes, counts[..., None])
    out = _loss(labels[:, None], features, protos_up)
    return out[0, 0]
